# Initial kernel scaffold; baseline (speedup 1.0000x reference)
import jax
import jax.numpy as jnp
from jax.experimental import pallas as pl


def kernel(x, edge_index, edge_feature, W_lin, b_lin, W_c0, b_c0, W_c1, b_c1, W_m0, b_m0, W_m1, b_m1):
    raise NotImplementedError("write your pallas kernel here")


# speedup vs baseline: 2.3115x; 2.3115x over previous
import jax
import jax.numpy as jnp
from jax.experimental import pallas as pl
from jax.experimental.pallas import tpu as pltpu

N = 10000
E = 320000
D_IN = 128
HID = 256
N_PAD = 10240
BLK = 1024


def _dense_body(x_ref, wlinT_ref, blin_ref, wc0T_ref, wmlp_ref, bmlp_ref,
                h1_ref, mlp_ref):
    xl = jnp.dot(x_ref[...], wlinT_ref[...],
                 preferred_element_type=jnp.float32) + blin_ref[...]
    h1_ref[...] = jnp.dot(xl, wc0T_ref[...],
                          preferred_element_type=jnp.float32)
    mlp_ref[...] = jnp.sum(xl * wmlp_ref[...], axis=1, keepdims=True) \
        + bmlp_ref[...]


def _dense(x_pad, W_linT, b_lin2, W_c0T, w_mlp2, b_mlp2):
    grid = N_PAD // BLK
    return pl.pallas_call(
        _dense_body,
        grid=(grid,),
        in_specs=[
            pl.BlockSpec((BLK, D_IN), lambda i: (i, 0)),
            pl.BlockSpec((D_IN, HID), lambda i: (0, 0)),
            pl.BlockSpec((1, HID), lambda i: (0, 0)),
            pl.BlockSpec((HID, D_IN), lambda i: (0, 0)),
            pl.BlockSpec((1, HID), lambda i: (0, 0)),
            pl.BlockSpec((1, 1), lambda i: (0, 0)),
        ],
        out_specs=[
            pl.BlockSpec((BLK, D_IN), lambda i: (i, 0)),
            pl.BlockSpec((BLK, 1), lambda i: (i, 0)),
        ],
        out_shape=[
            jax.ShapeDtypeStruct((N_PAD, D_IN), jnp.float32),
            jax.ShapeDtypeStruct((N_PAD, 1), jnp.float32),
        ],
    )(x_pad, W_linT, b_lin2, W_c0T, w_mlp2, b_mlp2)


def kernel(x, edge_index, edge_feature, W_lin, b_lin, W_c0, b_c0, W_c1, b_c1,
           W_m0, b_m0, W_m1, b_m1):
    src = edge_index[0]
    dst = edge_index[1]
    ew = edge_feature.mean(axis=1)

    w_mlp = (W_m1 @ W_m0)[0]
    b_mlp = (b_m1 + W_m1 @ b_m0)[0]

    x_pad = jnp.pad(x, ((0, N_PAD - N), (0, 0)))
    h1, mlp = _dense(x_pad, W_lin.T, b_lin[None, :], W_c0.T,
                     w_mlp[None, :], b_mlp[None, None])
    h1 = h1[:N]
    mlp = mlp[:N, 0]

    deg = jnp.zeros((N,), jnp.float32).at[dst].add(ew) + 1.0
    dis = jax.lax.rsqrt(deg)

    g1 = dis[:, None] * h1
    acc1 = jnp.zeros((N, D_IN), jnp.float32).at[dst].add(ew[:, None] * g1[src])
    xc1 = jax.nn.relu(dis[:, None] * (acc1 + g1) + b_c0[None, :])

    h2 = xc1 @ W_c1[0]
    g2 = dis * h2
    acc2 = jnp.zeros((N,), jnp.float32).at[dst].add(ew * g2[src])
    xc2 = jax.nn.relu(dis * (acc2 + g2) + b_c1[0])

    return xc2 + mlp


# speedup vs baseline: 14.7770x; 6.3929x over previous
import jax
import jax.numpy as jnp
from jax import lax
from jax.experimental import pallas as pl
from jax.experimental.pallas import tpu as pltpu
from jax.experimental.pallas import tpu_sc as plsc

N = 10000
E = 320000
D_IN = 128
HID = 256
N_PAD = 10240
BLK = 1024

NC = 2
NS = 16
NW = NC * NS
EPW = E // NW
EB = 80
NB = EPW // EB
BPS = 5
SB = NB // BPS
RPT = N_PAD // NS


def _mesh():
    return plsc.VectorSubcoreMesh(core_axis_name="c", subcore_axis_name="s")


_Z16 = lambda: jnp.zeros((16,), jnp.float32)

_GDN = lax.GatherDimensionNumbers(offset_dims=(), collapsed_slice_dims=(0,),
                                  start_index_map=(0,))


def _splat(w16, j):
    idx = jnp.full((16, 1), j, jnp.int32)
    return lax.gather(w16, idx, _GDN, slice_sizes=(1,),
                      mode=lax.GatherScatterMode.PROMISE_IN_BOUNDS)


def _dense_body(x_ref, wlinT_ref, blin_ref, wc0T_ref, wmlp_ref, bmlp_ref,
                dis_ref, g1_ref, mlp_ref):
    xl = jnp.dot(x_ref[...], wlinT_ref[...],
                 preferred_element_type=jnp.float32) + blin_ref[...]
    g1_ref[...] = dis_ref[...] * jnp.dot(xl, wc0T_ref[...],
                                         preferred_element_type=jnp.float32)
    mlp_ref[...] = jnp.sum(xl * wmlp_ref[...], axis=1, keepdims=True) \
        + bmlp_ref[...]


def _dense(x_pad, W_linT, b_lin2, W_c0T, w_mlp2, b_mlp2, dis2):
    return pl.pallas_call(
        _dense_body,
        grid=(N_PAD // BLK,),
        in_specs=[
            pl.BlockSpec((BLK, D_IN), lambda i: (i, 0)),
            pl.BlockSpec((D_IN, HID), lambda i: (0, 0)),
            pl.BlockSpec((1, HID), lambda i: (0, 0)),
            pl.BlockSpec((HID, D_IN), lambda i: (0, 0)),
            pl.BlockSpec((1, HID), lambda i: (0, 0)),
            pl.BlockSpec((1, 1), lambda i: (0, 0)),
            pl.BlockSpec((BLK, 1), lambda i: (i, 0)),
        ],
        out_specs=[
            pl.BlockSpec((BLK, D_IN), lambda i: (i, 0)),
            pl.BlockSpec((BLK, 1), lambda i: (i, 0)),
        ],
        out_shape=[
            jax.ShapeDtypeStruct((N_PAD, D_IN), jnp.float32),
            jax.ShapeDtypeStruct((N_PAD, 1), jnp.float32),
        ],
    )(x_pad, W_linT, b_lin2, W_c0T, w_mlp2, b_mlp2, dis2)


def _ew_body(xr_ref, m_ref, out_ref):
    out_ref[...] = jnp.dot(xr_ref[...], m_ref[...],
                           preferred_element_type=jnp.float32)


def _edge_mean(ef_rows, m):
    rows = ef_rows.shape[0]
    blk = rows // 8
    return pl.pallas_call(
        _ew_body,
        grid=(8,),
        in_specs=[
            pl.BlockSpec((blk, 128), lambda i: (i, 0)),
            pl.BlockSpec((128, 8), lambda i: (0, 0)),
        ],
        out_specs=pl.BlockSpec((blk, 8), lambda i: (i, 0)),
        out_shape=jax.ShapeDtypeStruct((rows, 8), jnp.float32),
    )(ef_rows, m)


def _post1_body(accp_ref, g1_ref, dis_ref, bc0_ref, wc1_ref, g2_ref):
    acc = accp_ref[0] + accp_ref[1]
    disb = dis_ref[...]
    xc1 = jnp.maximum(disb * (acc + g1_ref[...]) + bc0_ref[...], 0.0)
    h2 = jnp.sum(xc1 * wc1_ref[...], axis=1, keepdims=True)
    g2_ref[...] = disb * h2


def _post1(accp, h1, dis2, bc0, wc1):
    return pl.pallas_call(
        _post1_body,
        grid=(N_PAD // BLK,),
        in_specs=[
            pl.BlockSpec((NC, BLK, D_IN), lambda i: (0, i, 0)),
            pl.BlockSpec((BLK, D_IN), lambda i: (i, 0)),
            pl.BlockSpec((BLK, 1), lambda i: (i, 0)),
            pl.BlockSpec((1, D_IN), lambda i: (0, 0)),
            pl.BlockSpec((1, D_IN), lambda i: (0, 0)),
        ],
        out_specs=pl.BlockSpec((BLK, 1), lambda i: (i, 0)),
        out_shape=jax.ShapeDtypeStruct((N_PAD, 1), jnp.float32),
    )(accp, h1, dis2, bc0, wc1)


def _final_body(acc2p_ref, g2_ref, dis_ref, mlp_ref, bc1_ref, out_ref):
    a = acc2p_ref[0] + acc2p_ref[1]
    disb = dis_ref[...]
    xc2 = jnp.maximum(disb * (a + g2_ref[...]) + bc1_ref[...], 0.0)
    out_ref[...] = xc2 + mlp_ref[...]


def _final(acc2p3, g2, dis2, mlp, bc1):
    return pl.pallas_call(
        _final_body,
        grid=(N_PAD // BLK,),
        in_specs=[
            pl.BlockSpec((NC, BLK, 1), lambda i: (0, i, 0)),
            pl.BlockSpec((BLK, 1), lambda i: (i, 0)),
            pl.BlockSpec((BLK, 1), lambda i: (i, 0)),
            pl.BlockSpec((BLK, 1), lambda i: (i, 0)),
            pl.BlockSpec((1, 1), lambda i: (0, 0)),
        ],
        out_specs=pl.BlockSpec((BLK, 1), lambda i: (i, 0)),
        out_shape=jax.ShapeDtypeStruct((N_PAD, 1), jnp.float32),
    )(acc2p3, g2, dis2, mlp, bc1)


def _sc_deg_body(dst_hbm, ew_hbm, out_hbm, acc_sh, zbuf, dstv, ewv):
    c = lax.axis_index("c")
    s = lax.axis_index("s")
    wid = c * NS + s

    def zero_body(i, carry):
        zbuf[pl.ds(i * 16, 16)] = _Z16()
        return carry

    lax.fori_loop(0, RPT // 16, zero_body, None)
    pltpu.sync_copy(zbuf, acc_sh.at[pl.ds(s * RPT, RPT)])
    plsc.subcore_barrier()

    pltpu.sync_copy(dst_hbm.at[wid], dstv)
    pltpu.sync_copy(ew_hbm.at[wid], ewv)

    def batch_body(b, carry):
        pltpu.sync_copy(ewv.at[b], acc_sh.at[dstv.at[b]], add=True)
        return carry

    lax.fori_loop(0, NB, batch_body, None)
    plsc.subcore_barrier()
    pltpu.sync_copy(acc_sh.at[pl.ds(s * RPT, RPT)],
                    out_hbm.at[c, pl.ds(s * RPT, RPT)])


def _sc_deg(dst3, ew3):
    f = pl.kernel(
        _sc_deg_body,
        out_type=jax.ShapeDtypeStruct((NC, N_PAD), jnp.float32),
        mesh=_mesh(),
        scratch_types=[
            pltpu.VMEM_SHARED((N_PAD,), jnp.float32),
            pltpu.VMEM((RPT,), jnp.float32),
            pltpu.VMEM((NB, EB), jnp.int32),
            pltpu.VMEM((NB, EB), jnp.float32),
        ],
    )
    return f(dst3, ew3)


def _sc_l1_body(g1_hbm, src_hbm, dst_hbm, ew_hbm, out_hbm,
                acc_sh, srcv, dstv, ewv, rows):
    c = lax.axis_index("c")
    s = lax.axis_index("s")
    wid = c * NS + s

    def zero_body(i, carry):
        for q in range(D_IN // 16):
            rows[i, pl.ds(q * 16, 16)] = _Z16()
        return carry

    lax.fori_loop(0, EB, zero_body, None)
    for kk in range(RPT // EB):
        pltpu.sync_copy(rows, acc_sh.at[pl.ds(s * RPT + kk * EB, EB)])
    plsc.subcore_barrier()

    def super_body(sb, carry):
        pltpu.sync_copy(src_hbm.at[wid, sb], srcv)
        pltpu.sync_copy(dst_hbm.at[wid, sb], dstv)
        pltpu.sync_copy(ew_hbm.at[wid, sb], ewv)

        def batch_body(b, carry1):
            pltpu.sync_copy(g1_hbm.at[srcv.at[b]], rows)

            def group_body(gi, carry2):
                sl = pl.ds(gi * 16, 16)
                w16 = ewv[b, sl]
                for j in range(16):
                    wj = _splat(w16, j)
                    e = gi * 16 + j
                    for q in range(D_IN // 16):
                        qs = pl.ds(q * 16, 16)
                        rows[e, qs] = rows[e, qs] * wj
                return carry2

            lax.fori_loop(0, EB // 16, group_body, None)
            pltpu.sync_copy(rows, acc_sh.at[dstv.at[b]], add=True)
            return carry1

        lax.fori_loop(0, BPS, batch_body, None)
        return carry

    lax.fori_loop(0, SB, super_body, None)
    plsc.subcore_barrier()
    pltpu.sync_copy(acc_sh.at[pl.ds(s * RPT, RPT)],
                    out_hbm.at[c, pl.ds(s * RPT, RPT)])


def _sc_l1(g1, src4, dst4, ew4):
    f = pl.kernel(
        _sc_l1_body,
        out_type=jax.ShapeDtypeStruct((NC, N_PAD, D_IN), jnp.float32),
        mesh=_mesh(),
        scratch_types=[
            pltpu.VMEM_SHARED((N_PAD, D_IN), jnp.float32),
            pltpu.VMEM((BPS, EB), jnp.int32),
            pltpu.VMEM((BPS, EB), jnp.int32),
            pltpu.VMEM((BPS, EB), jnp.float32),
            pltpu.VMEM((EB, D_IN), jnp.float32),
        ],
    )
    return f(g1, src4, dst4, ew4)


def _sc_l2_body(g2_hbm, src_hbm, dst_hbm, ew_hbm, out_hbm,
                acc_sh, zbuf, srcv, dstv, ewv, gvals, valv):
    c = lax.axis_index("c")
    s = lax.axis_index("s")
    wid = c * NS + s

    def zero_body(i, carry):
        zbuf[pl.ds(i * 16, 16)] = _Z16()
        return carry

    lax.fori_loop(0, RPT // 16, zero_body, None)
    pltpu.sync_copy(zbuf, acc_sh.at[pl.ds(s * RPT, RPT)])
    plsc.subcore_barrier()

    pltpu.sync_copy(src_hbm.at[wid], srcv)
    pltpu.sync_copy(dst_hbm.at[wid], dstv)
    pltpu.sync_copy(ew_hbm.at[wid], ewv)

    def batch_body(b, carry):
        pltpu.sync_copy(g2_hbm.at[srcv.at[b]], gvals)

        def group_body(gi, carry2):
            sl = pl.ds(gi * 16, 16)
            valv[sl] = gvals[sl] * ewv[b, sl]
            return carry2

        lax.fori_loop(0, EB // 16, group_body, None)
        pltpu.sync_copy(valv, acc_sh.at[dstv.at[b]], add=True)
        return carry

    lax.fori_loop(0, NB, batch_body, None)
    plsc.subcore_barrier()
    pltpu.sync_copy(acc_sh.at[pl.ds(s * RPT, RPT)],
                    out_hbm.at[c, pl.ds(s * RPT, RPT)])


def _sc_l2(g2, src3, dst3, ew3):
    f = pl.kernel(
        _sc_l2_body,
        out_type=jax.ShapeDtypeStruct((NC, N_PAD), jnp.float32),
        mesh=_mesh(),
        scratch_types=[
            pltpu.VMEM_SHARED((N_PAD,), jnp.float32),
            pltpu.VMEM((RPT,), jnp.float32),
            pltpu.VMEM((NB, EB), jnp.int32),
            pltpu.VMEM((NB, EB), jnp.int32),
            pltpu.VMEM((NB, EB), jnp.float32),
            pltpu.VMEM((EB,), jnp.float32),
            pltpu.VMEM((EB,), jnp.float32),
        ],
    )
    return f(g2, src3, dst3, ew3)


def kernel(x, edge_index, edge_feature, W_lin, b_lin, W_c0, b_c0, W_c1, b_c1,
           W_m0, b_m0, W_m1, b_m1):
    src3 = edge_index[0].reshape(NW, NB, EB)
    dst3 = edge_index[1].reshape(NW, NB, EB)

    m = jnp.kron(jnp.eye(8, dtype=jnp.float32),
                 jnp.full((16, 1), 1.0 / 16, jnp.float32))
    ew3 = _edge_mean(edge_feature.reshape(E // 8, 128), m).reshape(NW, NB, EB)

    w_mlp = (W_m1 @ W_m0)[0]
    b_mlp = (b_m1 + W_m1 @ b_m0)[0]

    degp = _sc_deg(dst3, ew3)
    dis = lax.rsqrt(degp[0] + degp[1] + 1.0)
    dis2 = dis[:, None]

    x_pad = jnp.pad(x, ((0, N_PAD - N), (0, 0)))
    g1, mlp = _dense(x_pad, W_lin.T, b_lin[None, :], W_c0.T,
                     w_mlp[None, :], b_mlp[None, None], dis2)

    acc1p = _sc_l1(g1, src3.reshape(NW, SB, BPS, EB),
                   dst3.reshape(NW, SB, BPS, EB),
                   ew3.reshape(NW, SB, BPS, EB))
    g2 = _post1(acc1p, g1, dis2, b_c0[None, :], W_c1)

    acc2p = _sc_l2(g2.reshape(N_PAD), src3, dst3, ew3)

    out = _final(acc2p.reshape(NC, N_PAD, 1), g2, dis2, mlp,
                 b_c1[None, :])
    return out[:N, 0]
